# baseline (speedup 1.0000x reference)
import jax
import jax.numpy as jnp
from jax.experimental import pallas as pl
from jax.experimental.pallas import tpu as pltpu

_GAMMA = 2.0
_ALPHA = 0.25
_THR_POS = 0.05
_THR_NEG = 0.02

_B, _H, _W, _NB = 16, 128, 128, 1024
_N_PIX = _B * _H * _W
_RP = _B * _H
_RB = _B * _NB // 32


def _roll(x, shift):
    return jnp.roll(x, shift, axis=1)


def _iota2(shape, dim):
    return jax.lax.broadcasted_iota(jnp.int32, shape, dim)


def _loss_kernel(cl_ref, re_ref, gr_ref, gt_ref, cf_ref, op_ref, bb_ref,
                 br_ref, gbt_ref, out_ref):
    f32 = jnp.float32

    T = gt_ref[...].astype(f32)

    X = cl_ref[...]
    Xs = _roll(X, -1)
    lse = jnp.maximum(X, Xs) + jnp.log1p(jnp.exp(-jnp.abs(X - Xs)))
    E2 = (_iota2((_W, 2 * _W), 1) == 2 * _iota2((_W, 2 * _W), 0)).astype(
        jnp.bfloat16)
    T2 = jnp.dot(T.astype(jnp.bfloat16), E2, preferred_element_type=f32)
    xt = jnp.where(T2 >= 0.5, X, Xs)
    logpt = xt - lse
    pt = jnp.exp(logpt)
    at = jnp.where(T2 >= 0.5, _ALPHA, 1.0 - _ALPHA)
    om = 1.0 - pt
    term = -at * om * om * logpt
    rpn0_s = jnp.sum(jnp.where(_iota2((_RP, 2 * _W), 1) % 2 == 0, term, 0.0))

    R = re_ref[...]
    G = gr_ref[...]
    E4 = (_iota2((_W, 4 * _W), 1) == 4 * _iota2((_W, 4 * _W), 0)).astype(
        jnp.bfloat16)
    T4 = jnp.dot(T.astype(jnp.bfloat16), E4, preferred_element_type=f32)
    mn = jnp.minimum(R, G)
    s = mn + _roll(mn, -2)
    inter = s * _roll(s, -1)
    sg = G + _roll(G, -2)
    ga = sg * _roll(sg, -1)
    sr = R + _roll(R, -2)
    ra = sr * _roll(sr, -1)
    union = ga + ra - inter + 1e-7
    iou = (inter + 1.0) / (union + 1.0)
    rpn1_n = jnp.sum(jnp.where(T4 >= 0.5, 1.0 - iou, 0.0))
    rpn1_d = jnp.sum(T)

    Brr = br_ref[...]
    Bbb = bb_ref[...]
    Gv = gbt_ref[...]
    lm = _iota2((_RB, 128), 1) % 4
    lo = lm < 2

    eG = _roll(Gv, -2) - Gv
    areaA = jnp.maximum(eG, 0.0) * jnp.maximum(_roll(eG, -1), 0.0)

    c = jnp.where(lo, jnp.maximum(Brr, Gv), jnp.minimum(Brr, Gv))
    wh = jnp.maximum(_roll(c, -2) - c, 0.0)
    inter_b = wh * _roll(wh, -1)
    eB = _roll(Brr, -2) - Brr
    areaB = jnp.maximum(eB, 0.0) * jnp.maximum(_roll(eB, -1), 0.0)
    union_b = areaA + areaB - inter_b + 1e-7
    iou4 = inter_b / jnp.maximum(union_b, 1e-12)

    cb = jnp.where(lo, jnp.maximum(Bbb, Gv), jnp.minimum(Bbb, Gv))
    whb = jnp.maximum(_roll(cb, -2) - cb, 0.0)
    inter_bb = whb * _roll(whb, -1)
    eBB = _roll(Bbb, -2) - Bbb
    areaBB = jnp.maximum(eBB, 0.0) * jnp.maximum(_roll(eBB, -1), 0.0)
    union_bb = areaA + areaBB - inter_bb + 1.0
    iou_bb4 = inter_bb / jnp.maximum(union_bb, 1e-12)

    pos4 = jnp.logical_and(iou4 >= _THR_POS, lm == 0)
    q_bb = jnp.where(pos4, 1.0 - iou_bb4, 0.0)

    K4 = (_iota2((128, 32), 0) == 4 * _iota2((128, 32), 1)).astype(f32)
    iou_d = jnp.dot(iou4, K4, preferred_element_type=f32)
    pos_d = (iou_d >= _THR_POS).astype(f32)
    neg_d = (iou_d < _THR_NEG).astype(f32)

    xop = op_ref[...]
    bce = (jnp.maximum(xop, 0.0) - xop * iou_d
           + jnp.log1p(jnp.exp(-jnp.abs(xop))))

    C = cf_ref[...]
    Cs = _roll(C, -1)
    lseE = jnp.maximum(C, Cs) + jnp.log1p(jnp.exp(-jnp.abs(C - Cs)))
    lseF = jnp.where(_iota2((_RB, 256), 1) % 2 == 0, lseE, _roll(lseE, 1))
    nl = lseF - C
    rowc = _iota2((256, 32), 0)
    colc8 = _iota2((256, 32), 1)
    K0 = (rowc == 8 * colc8).astype(f32)
    K1 = (rowc == 8 * colc8 + 1).astype(f32)
    Kw = ((rowc == 8 * colc8 + 3) | (rowc == 8 * colc8 + 5)
          | (rowc == 8 * colc8 + 7)).astype(f32)
    U = jnp.dot(nl, K0, preferred_element_type=f32)
    V = jnp.dot(nl, K1, preferred_element_type=f32)
    Wn = jnp.dot(nl, Kw, preferred_element_type=f32)

    Asel = (_iota2((_B, _RB), 1) // 32 == _iota2((_B, _RB), 0)).astype(f32)

    def seg(q):
        rs = jnp.sum(q, axis=1, keepdims=True)
        return jnp.dot(Asel, rs, preferred_element_type=f32)

    pn = seg(pos_d)
    nn = seg(neg_d)
    s_op = seg(bce * pos_d)
    s_cfpos = seg(U * pos_d)
    s_cfnegb = seg(V * neg_d)
    s_cfneg = seg(Wn * pos_d)
    s_bb = seg(q_bb)

    pnp = pn > 0.0
    loss_op = jnp.where(pnp, s_op / jnp.maximum(pn, 1.0), 0.0)
    loss_cf_pos = jnp.where(pnp, s_cfpos / jnp.maximum(pn, 1.0), 0.0)
    loss_cf_negb = jnp.where(nn > 0.0, s_cfnegb / jnp.maximum(nn, 1.0), 0.0)
    loss_cf_neg = jnp.where(pnp, s_cfneg / jnp.maximum(3.0 * pn, 1.0), 0.0)
    loss_bb = jnp.where(pnp, s_bb / jnp.maximum(pn, 1.0), 0.0)
    loss_i = jnp.where(
        pnp, loss_cf_pos + loss_cf_negb + loss_cf_neg + loss_bb + loss_op, 0.0)

    rpn0 = rpn0_s / float(_N_PIX)
    rpn1 = jnp.where(rpn1_d > 0.0, rpn1_n / jnp.maximum(rpn1_d, 1.0), 0.0)
    rcnn = jnp.sum(loss_i) / float(_B)
    total_pos = jnp.sum(pn)

    out_ref[0, 0] = rpn0 + rpn1 + rcnn
    out_ref[0, 1] = rpn0
    out_ref[0, 2] = rpn1
    out_ref[0, 3] = rcnn
    out_ref[0, 4] = total_pos
    for i in range(5, 8):
        out_ref[0, i] = 0.0


def kernel(cl, re, cf, op, bb, br, gb, gr, gt):
    clr = cl.reshape(_RP, 2 * _W)
    rer = re.reshape(_RP, 4 * _W)
    grr = gr.reshape(_RP, 4 * _W)
    gtr = gt.reshape(_RP, _W)
    cfr = cf.reshape(_RB, 256)
    opr = op.reshape(_RB, 32)
    bbr = bb.reshape(_RB, 128)
    brr = br.reshape(_RB, 128)
    gbt = jnp.repeat(jnp.tile(gb, (1, 32)), 32, axis=0)

    out = pl.pallas_call(
        _loss_kernel,
        out_specs=pl.BlockSpec(memory_space=pltpu.SMEM),
        out_shape=jax.ShapeDtypeStruct((1, 8), jnp.float32),
    )(clr, rer, grr, gtr, cfr, opr, bbr, brr, gbt)

    return (out[0, 0], out[0, 1], out[0, 2], out[0, 3], out[0, 4])


# speedup vs baseline: 4.1657x; 4.1657x over previous
import jax
import jax.numpy as jnp
from jax.experimental import pallas as pl
from jax.experimental.pallas import tpu as pltpu

_GAMMA = 2.0
_ALPHA = 0.25
_THR_POS = 0.05
_THR_NEG = 0.02

_B, _H, _W, _NB = 16, 128, 128, 1024
_N_PIX = _B * _H * _W


def _rollr(x, shift):
    return jnp.roll(x, shift, axis=0)


def _roll(x, shift):
    return jnp.roll(x, shift, axis=1)


def _iota2(shape, dim):
    return jax.lax.broadcasted_iota(jnp.int32, shape, dim)


def _loss_kernel(cl_ref, re_ref, gr_ref, gt_ref, cf_ref, op_ref, bb_ref,
                 br_ref, gbt_ref, out_ref):
    b = pl.program_id(0)
    f32 = jnp.float32
    bf16 = jnp.bfloat16

    @pl.when(b == 0)
    def _init():
        for i in range(8):
            out_ref[0, i] = 0.0

    T = gt_ref[0].astype(f32)
    Tb = T.astype(bf16)

    X = cl_ref[0]
    Xs = _rollr(X, -1)
    lse = jnp.maximum(X, Xs) + jnp.log1p(jnp.exp(-jnp.abs(X - Xs)))
    E2 = (_iota2((2 * _H, _H), 0) == 2 * _iota2((2 * _H, _H), 1)).astype(bf16)
    T2 = jnp.dot(E2, Tb, preferred_element_type=f32)
    xt = jnp.where(T2 >= 0.5, X, Xs)
    logpt = xt - lse
    pt = jnp.exp(logpt)
    at = jnp.where(T2 >= 0.5, _ALPHA, 1.0 - _ALPHA)
    om = 1.0 - pt
    term = -at * om * om * logpt
    rpn0_s = jnp.sum(jnp.where(_iota2((2 * _H, _W), 0) % 2 == 0, term, 0.0))

    R = re_ref[0]
    G = gr_ref[0]
    E4 = (_iota2((4 * _H, _H), 0) == 4 * _iota2((4 * _H, _H), 1)).astype(bf16)
    T4 = jnp.dot(E4, Tb, preferred_element_type=f32)
    mn = jnp.minimum(R, G)
    s = mn + _rollr(mn, -2)
    inter = s * _rollr(s, -1)
    sg = G + _rollr(G, -2)
    ga = sg * _rollr(sg, -1)
    sr = R + _rollr(R, -2)
    ra = sr * _rollr(sr, -1)
    union = ga + ra - inter + 1e-7
    iou = (inter + 1.0) / (union + 1.0)
    rpn1_n = jnp.sum(jnp.where(T4 >= 0.5, 1.0 - iou, 0.0))
    rpn1_d = jnp.sum(T)

    Brr = br_ref[0]
    Bbb = bb_ref[0]
    Gv = gbt_ref[0]
    lm = _iota2((32, 128), 1) % 4
    lo = lm < 2

    eG = _roll(Gv, -2) - Gv
    areaA = jnp.maximum(eG, 0.0) * jnp.maximum(_roll(eG, -1), 0.0)

    c = jnp.where(lo, jnp.maximum(Brr, Gv), jnp.minimum(Brr, Gv))
    wh = jnp.maximum(_roll(c, -2) - c, 0.0)
    inter_b = wh * _roll(wh, -1)
    eB = _roll(Brr, -2) - Brr
    areaB = jnp.maximum(eB, 0.0) * jnp.maximum(_roll(eB, -1), 0.0)
    union_b = areaA + areaB - inter_b + 1e-7
    iou4 = inter_b / jnp.maximum(union_b, 1e-12)

    cb = jnp.where(lo, jnp.maximum(Bbb, Gv), jnp.minimum(Bbb, Gv))
    whb = jnp.maximum(_roll(cb, -2) - cb, 0.0)
    inter_bb = whb * _roll(whb, -1)
    eBB = _roll(Bbb, -2) - Bbb
    areaBB = jnp.maximum(eBB, 0.0) * jnp.maximum(_roll(eBB, -1), 0.0)
    union_bb = areaA + areaBB - inter_bb + 1.0
    iou_bb4 = inter_bb / jnp.maximum(union_bb, 1e-12)

    pos4 = jnp.logical_and(iou4 >= _THR_POS, lm == 0)
    s_bb = jnp.sum(jnp.where(pos4, 1.0 - iou_bb4, 0.0))

    K4 = (_iota2((128, 32), 0) == 4 * _iota2((128, 32), 1)).astype(f32)
    iou_d = jnp.dot(iou4, K4, preferred_element_type=f32)
    pos_d = (iou_d >= _THR_POS).astype(f32)
    neg_d = (iou_d < _THR_NEG).astype(f32)
    pn = jnp.sum(pos_d)
    nn = jnp.sum(neg_d)

    xop = op_ref[0]
    bce = (jnp.maximum(xop, 0.0) - xop * iou_d
           + jnp.log1p(jnp.exp(-jnp.abs(xop))))
    s_op = jnp.sum(bce * pos_d)

    C = cf_ref[0]
    Cs = _roll(C, -1)
    lseE = jnp.maximum(C, Cs) + jnp.log1p(jnp.exp(-jnp.abs(C - Cs)))
    lseF = jnp.where(_iota2((32, 256), 1) % 2 == 0, lseE, _roll(lseE, 1))
    nl = lseF - C
    rowc = _iota2((256, 32), 0)
    colc8 = _iota2((256, 32), 1)
    K0 = (rowc == 8 * colc8).astype(f32)
    K1 = (rowc == 8 * colc8 + 1).astype(f32)
    Kw = ((rowc == 8 * colc8 + 3) | (rowc == 8 * colc8 + 5)
          | (rowc == 8 * colc8 + 7)).astype(f32)
    U = jnp.dot(nl, K0, preferred_element_type=f32)
    V = jnp.dot(nl, K1, preferred_element_type=f32)
    Wn = jnp.dot(nl, Kw, preferred_element_type=f32)
    s_cfpos = jnp.sum(U * pos_d)
    s_cfnegb = jnp.sum(V * neg_d)
    s_cfneg = jnp.sum(Wn * pos_d)

    pnp = pn > 0.0
    loss_op = jnp.where(pnp, s_op / jnp.maximum(pn, 1.0), 0.0)
    loss_cf_pos = jnp.where(pnp, s_cfpos / jnp.maximum(pn, 1.0), 0.0)
    loss_cf_negb = jnp.where(nn > 0.0, s_cfnegb / jnp.maximum(nn, 1.0), 0.0)
    loss_cf_neg = jnp.where(pnp, s_cfneg / jnp.maximum(3.0 * pn, 1.0), 0.0)
    loss_bb = jnp.where(pnp, s_bb / jnp.maximum(pn, 1.0), 0.0)
    loss_i = jnp.where(
        pnp, loss_cf_pos + loss_cf_negb + loss_cf_neg + loss_bb + loss_op, 0.0)

    out_ref[0, 0] = out_ref[0, 0] + rpn0_s
    out_ref[0, 1] = out_ref[0, 1] + rpn1_n
    out_ref[0, 2] = out_ref[0, 2] + rpn1_d
    out_ref[0, 3] = out_ref[0, 3] + loss_i
    out_ref[0, 4] = out_ref[0, 4] + pn

    @pl.when(b == _B - 1)
    def _fin():
        a0 = out_ref[0, 0]
        a1 = out_ref[0, 1]
        a2 = out_ref[0, 2]
        a3 = out_ref[0, 3]
        rpn0 = a0 / float(_N_PIX)
        rpn1 = jnp.where(a2 > 0.0, a1 / jnp.maximum(a2, 1.0), 0.0)
        rcnn = a3 / float(_B)
        out_ref[0, 0] = rpn0 + rpn1 + rcnn
        out_ref[0, 1] = rpn0
        out_ref[0, 2] = rpn1
        out_ref[0, 3] = rcnn


def kernel(cl, re, cf, op, bb, br, gb, gr, gt):
    clv = cl.transpose(0, 1, 3, 2).reshape(_B, 2 * _H, _W)
    rev = re.transpose(0, 1, 3, 2).reshape(_B, 4 * _H, _W)
    grv = gr.transpose(0, 1, 3, 2).reshape(_B, 4 * _H, _W)
    cfr = cf.reshape(_B, 32, 256)
    opr = op.reshape(_B, 32, 32)
    bbr = bb.reshape(_B, 32, 128)
    brr = br.reshape(_B, 32, 128)
    gbt = jnp.tile(gb, (1, 32)).reshape(_B, 1, 128)

    out = pl.pallas_call(
        _loss_kernel,
        grid=(_B,),
        in_specs=[
            pl.BlockSpec((1, 2 * _H, _W), lambda b: (b, 0, 0)),
            pl.BlockSpec((1, 4 * _H, _W), lambda b: (b, 0, 0)),
            pl.BlockSpec((1, 4 * _H, _W), lambda b: (b, 0, 0)),
            pl.BlockSpec((1, _H, _W), lambda b: (b, 0, 0)),
            pl.BlockSpec((1, 32, 256), lambda b: (b, 0, 0)),
            pl.BlockSpec((1, 32, 32), lambda b: (b, 0, 0)),
            pl.BlockSpec((1, 32, 128), lambda b: (b, 0, 0)),
            pl.BlockSpec((1, 32, 128), lambda b: (b, 0, 0)),
            pl.BlockSpec((1, 1, 128), lambda b: (b, 0, 0)),
        ],
        out_specs=pl.BlockSpec((1, 8), lambda b: (0, 0),
                               memory_space=pltpu.SMEM),
        out_shape=jax.ShapeDtypeStruct((1, 8), jnp.float32),
        compiler_params=pltpu.CompilerParams(
            dimension_semantics=("arbitrary",)),
    )(clv, rev, grv, gt, cfr, opr, bbr, brr, gbt)

    return (out[0, 0], out[0, 1], out[0, 2], out[0, 3], out[0, 4])


# speedup vs baseline: 6.6938x; 1.6069x over previous
import jax
import jax.numpy as jnp
from jax.experimental import pallas as pl
from jax.experimental.pallas import tpu as pltpu

_GAMMA = 2.0
_ALPHA = 0.25
_THR_POS = 0.05
_THR_NEG = 0.02

_B, _H, _W, _NB = 16, 128, 128, 1024
_N_PIX = _B * _H * _W


def _rollr(x, shift):
    return jnp.roll(x, shift, axis=0)


def _iota2(shape, dim):
    return jax.lax.broadcasted_iota(jnp.int32, shape, dim)


def _loss_kernel(cl_ref, re_ref, gr_ref, gt_ref, cf_ref, op_ref, bb_ref,
                 br_ref, gbc_ref, out_ref):
    b = pl.program_id(0)
    f32 = jnp.float32
    bf16 = jnp.bfloat16

    @pl.when(b == 0)
    def _init():
        for i in range(8):
            out_ref[0, i] = 0.0

    T = gt_ref[0].astype(f32)
    Tb = T.astype(bf16)

    X = cl_ref[0]
    Xs = _rollr(X, -1)
    lse = jnp.maximum(X, Xs) + jnp.log1p(jnp.exp(-jnp.abs(X - Xs)))
    E2 = (_iota2((2 * _H, _H), 0) == 2 * _iota2((2 * _H, _H), 1)).astype(bf16)
    T2 = jnp.dot(E2, Tb, preferred_element_type=f32)
    xt = jnp.where(T2 >= 0.5, X, Xs)
    logpt = xt - lse
    pt = jnp.exp(logpt)
    at = jnp.where(T2 >= 0.5, _ALPHA, 1.0 - _ALPHA)
    om = 1.0 - pt
    term = -at * om * om * logpt
    rpn0_s = jnp.sum(jnp.where(_iota2((2 * _H, _W), 0) % 2 == 0, term, 0.0))

    R = re_ref[0]
    G = gr_ref[0]
    E4 = (_iota2((4 * _H, _H), 0) == 4 * _iota2((4 * _H, _H), 1)).astype(bf16)
    T4 = jnp.dot(E4, Tb, preferred_element_type=f32)
    mn = jnp.minimum(R, G)
    s = mn + _rollr(mn, -2)
    inter = s * _rollr(s, -1)
    sg = G + _rollr(G, -2)
    ga = sg * _rollr(sg, -1)
    sr = R + _rollr(R, -2)
    ra = sr * _rollr(sr, -1)
    union = ga + ra - inter + 1e-7
    iou = (inter + 1.0) / (union + 1.0)
    rpn1_n = jnp.sum(jnp.where(T4 >= 0.5, 1.0 - iou, 0.0))
    rpn1_d = jnp.sum(T)

    Brr = br_ref[0]
    Bbb = bb_ref[0]
    Gc = gbc_ref[0]
    rm = _iota2((32, 128), 0) % 4
    rlo = rm < 2

    eG = _rollr(Gc, -2) - Gc
    areaA = jnp.maximum(eG, 0.0) * jnp.maximum(_rollr(eG, -1), 0.0)

    c = jnp.where(rlo, jnp.maximum(Brr, Gc), jnp.minimum(Brr, Gc))
    wh = jnp.maximum(_rollr(c, -2) - c, 0.0)
    inter_b = wh * _rollr(wh, -1)
    eB = _rollr(Brr, -2) - Brr
    areaB = jnp.maximum(eB, 0.0) * jnp.maximum(_rollr(eB, -1), 0.0)
    union_b = areaA + areaB - inter_b + 1e-7
    iou_r = inter_b / jnp.maximum(union_b, 1e-12)

    cb = jnp.where(rlo, jnp.maximum(Bbb, Gc), jnp.minimum(Bbb, Gc))
    whb = jnp.maximum(_rollr(cb, -2) - cb, 0.0)
    inter_bb = whb * _rollr(whb, -1)
    eBB = _rollr(Bbb, -2) - Bbb
    areaBB = jnp.maximum(eBB, 0.0) * jnp.maximum(_rollr(eBB, -1), 0.0)
    union_bb = areaA + areaBB - inter_bb + 1.0
    iou_bb4 = inter_bb / jnp.maximum(union_bb, 1e-12)

    pos4 = jnp.logical_and(iou_r >= _THR_POS, rm == 0)
    s_bb = jnp.sum(jnp.where(pos4, 1.0 - iou_bb4, 0.0))

    Ksel = (_iota2((8, 32), 1) == 4 * _iota2((8, 32), 0)).astype(f32)
    iou_d = jnp.dot(Ksel, iou_r, preferred_element_type=f32)
    pos_d = (iou_d >= _THR_POS).astype(f32)
    neg_d = (iou_d < _THR_NEG).astype(f32)
    pn = jnp.sum(pos_d)
    nn = jnp.sum(neg_d)

    xop = op_ref[0]
    bce = (jnp.maximum(xop, 0.0) - xop * iou_d
           + jnp.log1p(jnp.exp(-jnp.abs(xop))))
    s_op = jnp.sum(bce * pos_d)

    C = cf_ref[0]
    Cs = _rollr(C, -1)
    lseE = jnp.maximum(C, Cs) + jnp.log1p(jnp.exp(-jnp.abs(C - Cs)))
    lseF = jnp.where(_iota2((64, 128), 0) % 2 == 0, lseE, _rollr(lseE, 1))
    nl = lseF - C
    rsel = _iota2((8, 64), 1)
    ksel = _iota2((8, 64), 0)
    U = jnp.dot((rsel == 2 * ksel).astype(f32), nl,
                preferred_element_type=f32)
    V = jnp.dot((rsel == 2 * ksel + 1).astype(f32), nl,
                preferred_element_type=f32)
    Wsel = ((rsel == 2 * ksel + 17) | (rsel == 2 * ksel + 33)
            | (rsel == 2 * ksel + 49)).astype(f32)
    Wn = jnp.dot(Wsel, nl, preferred_element_type=f32)
    s_cfpos = jnp.sum(U * pos_d)
    s_cfnegb = jnp.sum(V * neg_d)
    s_cfneg = jnp.sum(Wn * pos_d)

    pnp = pn > 0.0
    loss_op = jnp.where(pnp, s_op / jnp.maximum(pn, 1.0), 0.0)
    loss_cf_pos = jnp.where(pnp, s_cfpos / jnp.maximum(pn, 1.0), 0.0)
    loss_cf_negb = jnp.where(nn > 0.0, s_cfnegb / jnp.maximum(nn, 1.0), 0.0)
    loss_cf_neg = jnp.where(pnp, s_cfneg / jnp.maximum(3.0 * pn, 1.0), 0.0)
    loss_bb = jnp.where(pnp, s_bb / jnp.maximum(pn, 1.0), 0.0)
    loss_i = jnp.where(
        pnp, loss_cf_pos + loss_cf_negb + loss_cf_neg + loss_bb + loss_op, 0.0)

    out_ref[0, 0] = out_ref[0, 0] + rpn0_s
    out_ref[0, 1] = out_ref[0, 1] + rpn1_n
    out_ref[0, 2] = out_ref[0, 2] + rpn1_d
    out_ref[0, 3] = out_ref[0, 3] + loss_i
    out_ref[0, 4] = out_ref[0, 4] + pn

    @pl.when(b == _B - 1)
    def _fin():
        a0 = out_ref[0, 0]
        a1 = out_ref[0, 1]
        a2 = out_ref[0, 2]
        a3 = out_ref[0, 3]
        rpn0 = a0 / float(_N_PIX)
        rpn1 = jnp.where(a2 > 0.0, a1 / jnp.maximum(a2, 1.0), 0.0)
        rcnn = a3 / float(_B)
        out_ref[0, 0] = rpn0 + rpn1 + rcnn
        out_ref[0, 1] = rpn0
        out_ref[0, 2] = rpn1
        out_ref[0, 3] = rcnn


def kernel(cl, re, cf, op, bb, br, gb, gr, gt):
    clv = cl.transpose(0, 1, 3, 2).reshape(_B, 2 * _H, _W)
    rev = re.transpose(0, 1, 3, 2).reshape(_B, 4 * _H, _W)
    grv = gr.transpose(0, 1, 3, 2).reshape(_B, 4 * _H, _W)
    cfv = cf.reshape(_B, 8, 128, 4, 2).transpose(0, 3, 1, 4, 2) \
            .reshape(_B, 64, 128)
    opv = op.reshape(_B, 8, 128)
    bbv = bb.reshape(_B, 8, 128, 4).transpose(0, 1, 3, 2).reshape(_B, 32, 128)
    brv = br.reshape(_B, 8, 128, 4).transpose(0, 1, 3, 2).reshape(_B, 32, 128)
    gbc = jnp.tile(gb, (1, 8)).reshape(_B, 32, 1)

    out = pl.pallas_call(
        _loss_kernel,
        grid=(_B,),
        in_specs=[
            pl.BlockSpec((1, 2 * _H, _W), lambda b: (b, 0, 0)),
            pl.BlockSpec((1, 4 * _H, _W), lambda b: (b, 0, 0)),
            pl.BlockSpec((1, 4 * _H, _W), lambda b: (b, 0, 0)),
            pl.BlockSpec((1, _H, _W), lambda b: (b, 0, 0)),
            pl.BlockSpec((1, 64, 128), lambda b: (b, 0, 0)),
            pl.BlockSpec((1, 8, 128), lambda b: (b, 0, 0)),
            pl.BlockSpec((1, 32, 128), lambda b: (b, 0, 0)),
            pl.BlockSpec((1, 32, 128), lambda b: (b, 0, 0)),
            pl.BlockSpec((1, 32, 1), lambda b: (b, 0, 0)),
        ],
        out_specs=pl.BlockSpec((1, 8), lambda b: (0, 0),
                               memory_space=pltpu.SMEM),
        out_shape=jax.ShapeDtypeStruct((1, 8), jnp.float32),
        compiler_params=pltpu.CompilerParams(
            dimension_semantics=("arbitrary",)),
    )(clv, rev, grv, gt, cfv, opv, bbv, brv, gbc)

    return (out[0, 0], out[0, 1], out[0, 2], out[0, 3], out[0, 4])


# speedup vs baseline: 7.8990x; 1.1800x over previous
import jax
import jax.numpy as jnp
from jax.experimental import pallas as pl
from jax.experimental.pallas import tpu as pltpu

_GAMMA = 2.0
_ALPHA = 0.25
_THR_POS = 0.05
_THR_NEG = 0.02

_B, _H, _W, _NB = 16, 128, 128, 1024
_N_PIX = _B * _H * _W


def _loss_kernel(cl_ref, re_ref, gr_ref, gt_ref, cf_ref, op_ref, bb_ref,
                 br_ref, gbc_ref, out_ref):
    b = pl.program_id(0)
    f32 = jnp.float32

    @pl.when(b == 0)
    def _init():
        for i in range(8):
            out_ref[0, i] = 0.0

    T = gt_ref[0].astype(f32)

    x0 = cl_ref[0, 0::2, :]
    x1 = cl_ref[0, 1::2, :]
    lse = jnp.maximum(x0, x1) + jnp.log1p(jnp.exp(-jnp.abs(x0 - x1)))
    sel = T >= 0.5
    xt = jnp.where(sel, x0, x1)
    logpt = xt - lse
    pt = jnp.exp(logpt)
    at = jnp.where(sel, _ALPHA, 1.0 - _ALPHA)
    om = 1.0 - pt
    rpn0_s = jnp.sum(-at * om * om * logpt)

    r0 = re_ref[0, 0::4, :]
    r1 = re_ref[0, 1::4, :]
    r2 = re_ref[0, 2::4, :]
    r3 = re_ref[0, 3::4, :]
    g0 = gr_ref[0, 0::4, :]
    g1 = gr_ref[0, 1::4, :]
    g2 = gr_ref[0, 2::4, :]
    g3 = gr_ref[0, 3::4, :]
    inter = ((jnp.minimum(r0, g0) + jnp.minimum(r2, g2))
             * (jnp.minimum(r1, g1) + jnp.minimum(r3, g3)))
    ga = (g0 + g2) * (g1 + g3)
    ra = (r0 + r2) * (r1 + r3)
    union = ga + ra - inter + 1e-7
    iou = (inter + 1.0) / (union + 1.0)
    rpn1_n = jnp.sum((1.0 - iou) * T)
    rpn1_d = jnp.sum(T)

    gx1 = gbc_ref[0, 0::4, :]
    gy1 = gbc_ref[0, 1::4, :]
    gx2 = gbc_ref[0, 2::4, :]
    gy2 = gbc_ref[0, 3::4, :]
    areaA = (jnp.maximum(gx2 - gx1, 0.0)
             * jnp.maximum(gy2 - gy1, 0.0))

    def box_iou(bref, eps):
        bx1 = bref[0, 0::4, :]
        by1 = bref[0, 1::4, :]
        bx2 = bref[0, 2::4, :]
        by2 = bref[0, 3::4, :]
        whx = jnp.maximum(jnp.minimum(bx2, gx2) - jnp.maximum(bx1, gx1), 0.0)
        why = jnp.maximum(jnp.minimum(by2, gy2) - jnp.maximum(by1, gy1), 0.0)
        inter_ = whx * why
        areaB = (jnp.maximum(bx2 - bx1, 0.0)
                 * jnp.maximum(by2 - by1, 0.0))
        union_ = areaA + areaB - inter_ + eps
        return inter_ / jnp.maximum(union_, 1e-12)

    iou_d = box_iou(br_ref, 1e-7)
    iou_bb = box_iou(bb_ref, 1.0)
    pos_d = (iou_d >= _THR_POS).astype(f32)
    neg_d = (iou_d < _THR_NEG).astype(f32)
    pn = jnp.sum(pos_d)
    nn = jnp.sum(neg_d)
    s_bb = jnp.sum((1.0 - iou_bb) * pos_d)

    xop = op_ref[0]
    bce = (jnp.maximum(xop, 0.0) - xop * iou_d
           + jnp.log1p(jnp.exp(-jnp.abs(xop))))
    s_op = jnp.sum(bce * pos_d)

    Ca = cf_ref[0, 0::2, :]
    Cb = cf_ref[0, 1::2, :]
    lsec = jnp.maximum(Ca, Cb) + jnp.log1p(jnp.exp(-jnp.abs(Ca - Cb)))
    nl0 = lsec - Ca
    nl1 = lsec - Cb
    s_cfpos = jnp.sum(nl0[0:8] * pos_d)
    s_cfnegb = jnp.sum(nl1[0:8] * neg_d)
    s_cfneg = jnp.sum((nl1[8:16] + nl1[16:24] + nl1[24:32]) * pos_d)

    pnp = pn > 0.0
    loss_op = jnp.where(pnp, s_op / jnp.maximum(pn, 1.0), 0.0)
    loss_cf_pos = jnp.where(pnp, s_cfpos / jnp.maximum(pn, 1.0), 0.0)
    loss_cf_negb = jnp.where(nn > 0.0, s_cfnegb / jnp.maximum(nn, 1.0), 0.0)
    loss_cf_neg = jnp.where(pnp, s_cfneg / jnp.maximum(3.0 * pn, 1.0), 0.0)
    loss_bb = jnp.where(pnp, s_bb / jnp.maximum(pn, 1.0), 0.0)
    loss_i = jnp.where(
        pnp, loss_cf_pos + loss_cf_negb + loss_cf_neg + loss_bb + loss_op, 0.0)

    out_ref[0, 0] = out_ref[0, 0] + rpn0_s
    out_ref[0, 1] = out_ref[0, 1] + rpn1_n
    out_ref[0, 2] = out_ref[0, 2] + rpn1_d
    out_ref[0, 3] = out_ref[0, 3] + loss_i
    out_ref[0, 4] = out_ref[0, 4] + pn

    @pl.when(b == _B - 1)
    def _fin():
        a0 = out_ref[0, 0]
        a1 = out_ref[0, 1]
        a2 = out_ref[0, 2]
        a3 = out_ref[0, 3]
        rpn0 = a0 / float(_N_PIX)
        rpn1 = jnp.where(a2 > 0.0, a1 / jnp.maximum(a2, 1.0), 0.0)
        rcnn = a3 / float(_B)
        out_ref[0, 0] = rpn0 + rpn1 + rcnn
        out_ref[0, 1] = rpn0
        out_ref[0, 2] = rpn1
        out_ref[0, 3] = rcnn


def kernel(cl, re, cf, op, bb, br, gb, gr, gt):
    clv = cl.transpose(0, 1, 3, 2).reshape(_B, 2 * _H, _W)
    rev = re.transpose(0, 1, 3, 2).reshape(_B, 4 * _H, _W)
    grv = gr.transpose(0, 1, 3, 2).reshape(_B, 4 * _H, _W)
    cfv = cf.reshape(_B, 8, 128, 4, 2).transpose(0, 3, 1, 4, 2) \
            .reshape(_B, 64, 128)
    opv = op.reshape(_B, 8, 128)
    bbv = bb.reshape(_B, 8, 128, 4).transpose(0, 1, 3, 2).reshape(_B, 32, 128)
    brv = br.reshape(_B, 8, 128, 4).transpose(0, 1, 3, 2).reshape(_B, 32, 128)
    gbc = jnp.tile(gb, (1, 8)).reshape(_B, 32, 1)

    out = pl.pallas_call(
        _loss_kernel,
        grid=(_B,),
        in_specs=[
            pl.BlockSpec((1, 2 * _H, _W), lambda b: (b, 0, 0)),
            pl.BlockSpec((1, 4 * _H, _W), lambda b: (b, 0, 0)),
            pl.BlockSpec((1, 4 * _H, _W), lambda b: (b, 0, 0)),
            pl.BlockSpec((1, _H, _W), lambda b: (b, 0, 0)),
            pl.BlockSpec((1, 64, 128), lambda b: (b, 0, 0)),
            pl.BlockSpec((1, 8, 128), lambda b: (b, 0, 0)),
            pl.BlockSpec((1, 32, 128), lambda b: (b, 0, 0)),
            pl.BlockSpec((1, 32, 128), lambda b: (b, 0, 0)),
            pl.BlockSpec((1, 32, 1), lambda b: (b, 0, 0)),
        ],
        out_specs=pl.BlockSpec((1, 8), lambda b: (0, 0),
                               memory_space=pltpu.SMEM),
        out_shape=jax.ShapeDtypeStruct((1, 8), jnp.float32),
        compiler_params=pltpu.CompilerParams(
            dimension_semantics=("arbitrary",)),
    )(clv, rev, grv, gt, cfv, opv, bbv, brv, gbc)

    return (out[0, 0], out[0, 1], out[0, 2], out[0, 3], out[0, 4])


# speedup vs baseline: 11.0246x; 1.3957x over previous
import jax
import jax.numpy as jnp
from jax.experimental import pallas as pl
from jax.experimental.pallas import tpu as pltpu

_GAMMA = 2.0
_ALPHA = 0.25
_THR_POS = 0.05
_THR_NEG = 0.02

_B, _H, _W, _NB = 16, 128, 128, 1024
_N_PIX = _B * _H * _W
_BPS = 4
_STEPS = _B // _BPS


def _loss_kernel(cl_ref, re_ref, gr_ref, gt_ref, cf_ref, op_ref, bb_ref,
                 br_ref, gbc_ref, out_ref):
    g = pl.program_id(0)
    f32 = jnp.float32

    @pl.when(g == 0)
    def _init():
        for i in range(8):
            out_ref[0, i] = 0.0

    T = gt_ref[...].astype(f32)

    x0 = cl_ref[0::2, :]
    x1 = cl_ref[1::2, :]
    lse = jnp.maximum(x0, x1) + jnp.log1p(jnp.exp(-jnp.abs(x0 - x1)))
    sel = T >= 0.5
    xt = jnp.where(sel, x0, x1)
    logpt = xt - lse
    pt = jnp.exp(logpt)
    at = jnp.where(sel, _ALPHA, 1.0 - _ALPHA)
    om = 1.0 - pt
    rpn0_s = jnp.sum(-at * om * om * logpt)

    r0 = re_ref[0::4, :]
    r1 = re_ref[1::4, :]
    r2 = re_ref[2::4, :]
    r3 = re_ref[3::4, :]
    g0 = gr_ref[0::4, :]
    g1 = gr_ref[1::4, :]
    g2 = gr_ref[2::4, :]
    g3 = gr_ref[3::4, :]
    inter = ((jnp.minimum(r0, g0) + jnp.minimum(r2, g2))
             * (jnp.minimum(r1, g1) + jnp.minimum(r3, g3)))
    ga = (g0 + g2) * (g1 + g3)
    ra = (r0 + r2) * (r1 + r3)
    union = ga + ra - inter + 1e-7
    iou = (inter + 1.0) / (union + 1.0)
    rpn1_n = jnp.sum((1.0 - iou) * T)
    rpn1_d = jnp.sum(T)

    gx1 = gbc_ref[0::4, :]
    gy1 = gbc_ref[1::4, :]
    gx2 = gbc_ref[2::4, :]
    gy2 = gbc_ref[3::4, :]
    areaA = (jnp.maximum(gx2 - gx1, 0.0)
             * jnp.maximum(gy2 - gy1, 0.0))

    def box_iou(bref, eps):
        bx1 = bref[0::4, :]
        by1 = bref[1::4, :]
        bx2 = bref[2::4, :]
        by2 = bref[3::4, :]
        whx = jnp.maximum(jnp.minimum(bx2, gx2) - jnp.maximum(bx1, gx1), 0.0)
        why = jnp.maximum(jnp.minimum(by2, gy2) - jnp.maximum(by1, gy1), 0.0)
        inter_ = whx * why
        areaB = (jnp.maximum(bx2 - bx1, 0.0)
                 * jnp.maximum(by2 - by1, 0.0))
        union_ = areaA + areaB - inter_ + eps
        return inter_ / jnp.maximum(union_, 1e-12)

    iou_d = box_iou(br_ref, 1e-7)
    iou_bb = box_iou(bb_ref, 1.0)
    pos_d = (iou_d >= _THR_POS).astype(f32)
    neg_d = (iou_d < _THR_NEG).astype(f32)
    q_bb = (1.0 - iou_bb) * pos_d

    xop = op_ref[...]
    bce = (jnp.maximum(xop, 0.0) - xop * iou_d
           + jnp.log1p(jnp.exp(-jnp.abs(xop))))
    q_op = bce * pos_d

    Ca = cf_ref[0::2, :]
    Cb = cf_ref[1::2, :]
    lsec = jnp.maximum(Ca, Cb) + jnp.log1p(jnp.exp(-jnp.abs(Ca - Cb)))
    nl0 = lsec - Ca
    nl1 = lsec - Cb

    rcnn_s = jnp.float32(0.0)
    pn_tot = jnp.float32(0.0)
    for j in range(_BPS):
        r8 = slice(8 * j, 8 * (j + 1))
        r32 = slice(32 * j, 32 * j + 8)
        pos_j = pos_d[r8]
        neg_j = neg_d[r8]
        pn = jnp.sum(pos_j)
        nn = jnp.sum(neg_j)
        s_bb = jnp.sum(q_bb[r8])
        s_op = jnp.sum(q_op[r8])
        s_cfpos = jnp.sum(nl0[r32] * pos_j)
        s_cfnegb = jnp.sum(nl1[r32] * neg_j)
        s_cfneg = jnp.sum((nl1[32 * j + 8:32 * j + 16]
                           + nl1[32 * j + 16:32 * j + 24]
                           + nl1[32 * j + 24:32 * j + 32]) * pos_j)
        pnp = pn > 0.0
        l_op = jnp.where(pnp, s_op / jnp.maximum(pn, 1.0), 0.0)
        l_cfp = jnp.where(pnp, s_cfpos / jnp.maximum(pn, 1.0), 0.0)
        l_cfnb = jnp.where(nn > 0.0, s_cfnegb / jnp.maximum(nn, 1.0), 0.0)
        l_cfn = jnp.where(pnp, s_cfneg / jnp.maximum(3.0 * pn, 1.0), 0.0)
        l_bb = jnp.where(pnp, s_bb / jnp.maximum(pn, 1.0), 0.0)
        rcnn_s = rcnn_s + jnp.where(
            pnp, l_cfp + l_cfnb + l_cfn + l_bb + l_op, 0.0)
        pn_tot = pn_tot + pn

    out_ref[0, 0] = out_ref[0, 0] + rpn0_s
    out_ref[0, 1] = out_ref[0, 1] + rpn1_n
    out_ref[0, 2] = out_ref[0, 2] + rpn1_d
    out_ref[0, 3] = out_ref[0, 3] + rcnn_s
    out_ref[0, 4] = out_ref[0, 4] + pn_tot

    @pl.when(g == _STEPS - 1)
    def _fin():
        a0 = out_ref[0, 0]
        a1 = out_ref[0, 1]
        a2 = out_ref[0, 2]
        a3 = out_ref[0, 3]
        rpn0 = a0 / float(_N_PIX)
        rpn1 = jnp.where(a2 > 0.0, a1 / jnp.maximum(a2, 1.0), 0.0)
        rcnn = a3 / float(_B)
        out_ref[0, 0] = rpn0 + rpn1 + rcnn
        out_ref[0, 1] = rpn0
        out_ref[0, 2] = rpn1
        out_ref[0, 3] = rcnn


def kernel(cl, re, cf, op, bb, br, gb, gr, gt):
    clv = cl.transpose(0, 1, 3, 2).reshape(_B * 2 * _H, _W)
    rev = re.transpose(0, 1, 3, 2).reshape(_B * 4 * _H, _W)
    grv = gr.transpose(0, 1, 3, 2).reshape(_B * 4 * _H, _W)
    gtv = gt.reshape(_B * _H, _W)
    cfv = cf.reshape(_B, 8, 128, 4, 2).transpose(0, 3, 1, 4, 2) \
            .reshape(_B * 64, 128)
    opv = op.reshape(_B * 8, 128)
    bbv = bb.reshape(_B, 8, 128, 4).transpose(0, 1, 3, 2).reshape(_B * 32, 128)
    brv = br.reshape(_B, 8, 128, 4).transpose(0, 1, 3, 2).reshape(_B * 32, 128)
    gbc = jnp.tile(gb, (1, 8)).reshape(_B * 32, 1)

    out = pl.pallas_call(
        _loss_kernel,
        grid=(_STEPS,),
        in_specs=[
            pl.BlockSpec((_BPS * 2 * _H, _W), lambda g: (g, 0)),
            pl.BlockSpec((_BPS * 4 * _H, _W), lambda g: (g, 0)),
            pl.BlockSpec((_BPS * 4 * _H, _W), lambda g: (g, 0)),
            pl.BlockSpec((_BPS * _H, _W), lambda g: (g, 0)),
            pl.BlockSpec((_BPS * 64, 128), lambda g: (g, 0)),
            pl.BlockSpec((_BPS * 8, 128), lambda g: (g, 0)),
            pl.BlockSpec((_BPS * 32, 128), lambda g: (g, 0)),
            pl.BlockSpec((_BPS * 32, 128), lambda g: (g, 0)),
            pl.BlockSpec((_BPS * 32, 1), lambda g: (g, 0)),
        ],
        out_specs=pl.BlockSpec((1, 8), lambda g: (0, 0),
                               memory_space=pltpu.SMEM),
        out_shape=jax.ShapeDtypeStruct((1, 8), jnp.float32),
        compiler_params=pltpu.CompilerParams(
            dimension_semantics=("arbitrary",)),
    )(clv, rev, grv, gtv, cfv, opv, bbv, brv, gbc)

    return (out[0, 0], out[0, 1], out[0, 2], out[0, 3], out[0, 4])
